# Initial kernel scaffold; baseline (speedup 1.0000x reference)
import jax
import jax.numpy as jnp
from jax.experimental import pallas as pl


def kernel(x, edge_index, W1, a_src1, a_dst1, b1, gn1_w, gn1_b, gn1_ms, W2, a_src2, a_dst2, b2, gn2_w, gn2_b, gn2_ms, lin1_W, lin1_b, bn1_w, bn1_b, bn1_rm, bn1_rv, lin2_W, lin2_b):
    raise NotImplementedError("write your pallas kernel here")


# speedup vs baseline: 10.8703x; 10.8703x over previous
import functools
import jax
import jax.numpy as jnp
from jax.experimental import pallas as pl

_BLK_N = 1024
_BLK_E = 4096
_PREC = jax.lax.Precision.HIGHEST


def _ceil_div(a, b):
    return -(-a // b)


def _dense_body(x_ref, w_ref, pa_s_ref, pa_d_ref, h_ref, asrc_ref, adst_ref, esl_ref):
    h = jnp.dot(x_ref[...], w_ref[...], precision=_PREC)
    h_ref[...] = h
    asrc = jnp.dot(h, pa_s_ref[...], precision=_PREC)
    adst = jnp.dot(h, pa_d_ref[...], precision=_PREC)
    asrc_ref[...] = asrc
    adst_ref[...] = adst
    s = asrc + adst
    esl_ref[...] = jnp.where(s >= 0, s, 0.2 * s)


def _edge_logit_body(es_ref, ed_ref, t_ref, bmax_ref):
    s = es_ref[...] + ed_ref[...]
    t = jnp.where(s >= 0, s, 0.2 * s)
    t_ref[...] = t
    bmax_ref[...] = jnp.broadcast_to(jnp.max(t, axis=0, keepdims=True), bmax_ref.shape)


def _edge_msg_body(t_ref, hs_ref, gmax_ref, r_ref, ee_ref, msg_ref):
    ee = jnp.exp(t_ref[...] - gmax_ref[0, 0])
    ee_ref[...] = ee
    rep = jnp.dot(ee, r_ref[...], precision=_PREC)
    msg_ref[...] = hs_ref[...] * rep


def _elu(x):
    return jnp.where(x > 0, x, jnp.exp(x) - 1.0)


def _fin_body(acc_ref, den_ref, eslr_ref, h_ref, gmax_ref, b_ref,
              g_ref, psum_ref, *, n, blk):
    ees = jnp.exp(eslr_ref[...] - gmax_ref[0, 0])
    den = den_ref[...] + ees
    acc = acc_ref[...] + ees * h_ref[...]
    g = _elu(acc / (den + 1e-16) + b_ref[...])
    g_ref[...] = g
    i = pl.program_id(0)
    rows = jax.lax.broadcasted_iota(jnp.int32, (blk, 1), 0) + i * blk
    mask = rows < n
    psum_ref[...] = jnp.broadcast_to(
        jnp.sum(jnp.where(mask, g, 0.0), axis=0, keepdims=True), psum_ref.shape)


def _center_body(g_ref, mean_ref, ms_ref, c_ref, psq_ref, *, n, blk):
    c = g_ref[...] - ms_ref[...] * mean_ref[...]
    c_ref[...] = c
    i = pl.program_id(0)
    rows = jax.lax.broadcasted_iota(jnp.int32, (blk, 1), 0) + i * blk
    mask = rows < n
    cm = jnp.where(mask, c, 0.0)
    psq_ref[...] = jnp.broadcast_to(
        jnp.sum(cm * cm, axis=0, keepdims=True), psq_ref.shape)


def _scale_body(c_ref, var_ref, w_ref, b_ref, o_ref):
    o_ref[...] = w_ref[...] * c_ref[...] / jnp.sqrt(var_ref[...] + 1e-5) + b_ref[...]


def _head_body(hr_ref, w1_ref, b1_ref, bnw_ref, bnb_ref, bnrm_ref, bnrv_ref,
               w2_ref, b2_ref, o_ref):
    y = jnp.dot(hr_ref[...], w1_ref[...], precision=_PREC) + b1_ref[...]
    y = _elu(y)
    y = (y - bnrm_ref[...]) / jnp.sqrt(bnrv_ref[...] + 1e-5) * bnw_ref[...] + bnb_ref[...]
    o_ref[...] = jnp.dot(y, w2_ref[...], precision=_PREC) + b2_ref[...]


def _gat_layer(x_pad, n, src, dst_p, e_pad_to, W, a_s, a_d, b, gn_w, gn_b, gn_ms):
    npad, f = x_pad.shape
    heads, hid = a_s.shape
    hf = heads * hid
    nb = npad // _BLK_N
    eye = jnp.eye(heads, dtype=jnp.float32)
    pa_s = (a_s[:, :, None] * eye[:, None, :]).reshape(hf, heads)
    pa_d = (a_d[:, :, None] * eye[:, None, :]).reshape(hf, heads)
    rmat = jnp.repeat(eye, hid, axis=1)

    h, asrc, adst, esl = pl.pallas_call(
        _dense_body,
        grid=(nb,),
        in_specs=[
            pl.BlockSpec((_BLK_N, f), lambda i: (i, 0)),
            pl.BlockSpec((f, hf), lambda i: (0, 0)),
            pl.BlockSpec((hf, heads), lambda i: (0, 0)),
            pl.BlockSpec((hf, heads), lambda i: (0, 0)),
        ],
        out_specs=[
            pl.BlockSpec((_BLK_N, hf), lambda i: (i, 0)),
            pl.BlockSpec((_BLK_N, heads), lambda i: (i, 0)),
            pl.BlockSpec((_BLK_N, heads), lambda i: (i, 0)),
            pl.BlockSpec((_BLK_N, heads), lambda i: (i, 0)),
        ],
        out_shape=[
            jax.ShapeDtypeStruct((npad, hf), jnp.float32),
            jax.ShapeDtypeStruct((npad, heads), jnp.float32),
            jax.ShapeDtypeStruct((npad, heads), jnp.float32),
            jax.ShapeDtypeStruct((npad, heads), jnp.float32),
        ],
    )(x_pad, W, pa_s, pa_d)

    e = src.shape[0]
    ne = e_pad_to // _BLK_E
    es = jnp.pad(asrc[src], ((0, e_pad_to - e), (0, 0)), constant_values=-1e30)
    ed = jnp.pad(adst[dst_p[:e]], ((0, e_pad_to - e), (0, 0)), constant_values=-1e30)

    t, bmax = pl.pallas_call(
        _edge_logit_body,
        grid=(ne,),
        in_specs=[
            pl.BlockSpec((_BLK_E, heads), lambda i: (i, 0)),
            pl.BlockSpec((_BLK_E, heads), lambda i: (i, 0)),
        ],
        out_specs=[
            pl.BlockSpec((_BLK_E, heads), lambda i: (i, 0)),
            pl.BlockSpec((8, heads), lambda i: (i, 0)),
        ],
        out_shape=[
            jax.ShapeDtypeStruct((e_pad_to, heads), jnp.float32),
            jax.ShapeDtypeStruct((ne * 8, heads), jnp.float32),
        ],
    )(es, ed)

    gmax = jnp.maximum(jnp.max(bmax), jnp.max(esl[:n]))
    gmax = gmax.reshape(1, 1)

    hs = jnp.pad(h[src], ((0, e_pad_to - e), (0, 0)))

    ee, msg = pl.pallas_call(
        _edge_msg_body,
        grid=(ne,),
        in_specs=[
            pl.BlockSpec((_BLK_E, heads), lambda i: (i, 0)),
            pl.BlockSpec((_BLK_E, hf), lambda i: (i, 0)),
            pl.BlockSpec((1, 1), lambda i: (0, 0)),
            pl.BlockSpec((heads, hf), lambda i: (0, 0)),
        ],
        out_specs=[
            pl.BlockSpec((_BLK_E, heads), lambda i: (i, 0)),
            pl.BlockSpec((_BLK_E, hf), lambda i: (i, 0)),
        ],
        out_shape=[
            jax.ShapeDtypeStruct((e_pad_to, heads), jnp.float32),
            jax.ShapeDtypeStruct((e_pad_to, hf), jnp.float32),
        ],
    )(t, hs, gmax, rmat)

    denom = jax.ops.segment_sum(ee, dst_p, num_segments=n)
    accum = jax.ops.segment_sum(msg, dst_p, num_segments=n)

    den_rep = jnp.pad(jnp.repeat(denom, hid, axis=1), ((0, npad - n), (0, 0)))
    acc_pad = jnp.pad(accum, ((0, npad - n), (0, 0)))
    eslr = jnp.repeat(esl, hid, axis=1)

    g, psum = pl.pallas_call(
        functools.partial(_fin_body, n=n, blk=_BLK_N),
        grid=(nb,),
        in_specs=[
            pl.BlockSpec((_BLK_N, hf), lambda i: (i, 0)),
            pl.BlockSpec((_BLK_N, hf), lambda i: (i, 0)),
            pl.BlockSpec((_BLK_N, hf), lambda i: (i, 0)),
            pl.BlockSpec((_BLK_N, hf), lambda i: (i, 0)),
            pl.BlockSpec((1, 1), lambda i: (0, 0)),
            pl.BlockSpec((1, hf), lambda i: (0, 0)),
        ],
        out_specs=[
            pl.BlockSpec((_BLK_N, hf), lambda i: (i, 0)),
            pl.BlockSpec((8, hf), lambda i: (i, 0)),
        ],
        out_shape=[
            jax.ShapeDtypeStruct((npad, hf), jnp.float32),
            jax.ShapeDtypeStruct((nb * 8, hf), jnp.float32),
        ],
    )(acc_pad, den_rep, eslr, h, gmax, b.reshape(1, hf))

    mean = (jnp.sum(psum, axis=0, keepdims=True) / (8.0 * n))

    c, psq = pl.pallas_call(
        functools.partial(_center_body, n=n, blk=_BLK_N),
        grid=(nb,),
        in_specs=[
            pl.BlockSpec((_BLK_N, hf), lambda i: (i, 0)),
            pl.BlockSpec((1, hf), lambda i: (0, 0)),
            pl.BlockSpec((1, hf), lambda i: (0, 0)),
        ],
        out_specs=[
            pl.BlockSpec((_BLK_N, hf), lambda i: (i, 0)),
            pl.BlockSpec((8, hf), lambda i: (i, 0)),
        ],
        out_shape=[
            jax.ShapeDtypeStruct((npad, hf), jnp.float32),
            jax.ShapeDtypeStruct((nb * 8, hf), jnp.float32),
        ],
    )(g, mean, gn_ms.reshape(1, hf))

    var = (jnp.sum(psq, axis=0, keepdims=True) / (8.0 * n))

    out = pl.pallas_call(
        _scale_body,
        grid=(nb,),
        in_specs=[
            pl.BlockSpec((_BLK_N, hf), lambda i: (i, 0)),
            pl.BlockSpec((1, hf), lambda i: (0, 0)),
            pl.BlockSpec((1, hf), lambda i: (0, 0)),
            pl.BlockSpec((1, hf), lambda i: (0, 0)),
        ],
        out_specs=pl.BlockSpec((_BLK_N, hf), lambda i: (i, 0)),
        out_shape=jax.ShapeDtypeStruct((npad, hf), jnp.float32),
    )(c, var, gn_w.reshape(1, hf), gn_b.reshape(1, hf))

    return out


def kernel(x, edge_index, W1, a_src1, a_dst1, b1, gn1_w, gn1_b, gn1_ms,
           W2, a_src2, a_dst2, b2, gn2_w, gn2_b, gn2_ms,
           lin1_W, lin1_b, bn1_w, bn1_b, bn1_rm, bn1_rv, lin2_W, lin2_b):
    n = x.shape[0]
    e = edge_index.shape[1]
    npad = _ceil_div(n, _BLK_N) * _BLK_N
    e_pad_to = _ceil_div(e, _BLK_E) * _BLK_E

    src = edge_index[0]
    dst_p = jnp.pad(edge_index[1], (0, e_pad_to - e))

    x_pad = jnp.pad(x, ((0, npad - n), (0, 0)))
    h1 = _gat_layer(x_pad, n, src, dst_p, e_pad_to, W1, a_src1, a_dst1, b1,
                    gn1_w, gn1_b, gn1_ms)
    h2 = _gat_layer(h1, n, src, dst_p, e_pad_to, W2, a_src2, a_dst2, b2,
                    gn2_w, gn2_b, gn2_ms)

    lin1_in = lin1_W.shape[0]
    hid = lin1_W.shape[1]
    ncls = lin2_W.shape[1]
    hr = h2[:n].reshape(-1, lin1_in)
    rows = hr.shape[0]

    out = pl.pallas_call(
        _head_body,
        out_shape=jax.ShapeDtypeStruct((rows, ncls), jnp.float32),
    )(hr, lin1_W, lin1_b.reshape(1, hid), bn1_w.reshape(1, hid),
      bn1_b.reshape(1, hid), bn1_rm.reshape(1, hid), bn1_rv.reshape(1, hid),
      lin2_W, lin2_b.reshape(1, ncls))

    return out
